# Initial kernel scaffold; baseline (speedup 1.0000x reference)
import jax
import jax.numpy as jnp
from jax.experimental import pallas as pl


def kernel(features, enc0_Ws, enc0_Wn, enc0_b, enc1_Ws, enc1_Wn, enc1_b, enc2_Ws, enc2_Wn, enc2_b, ubend_Ws, ubend_Wn, ubend_b, dec0_Ws, dec0_Wn, dec0_b, dec1_Ws, dec1_Wn, dec1_b, dec2_Ws, dec2_Wn, dec2_b, edge_index_0, edge_index_1, edge_index_2, edge_index_3, cluster_1, cluster_2, cluster_3):
    raise NotImplementedError("write your pallas kernel here")


# speedup vs baseline: 1.1611x; 1.1611x over previous
import functools

import jax
import jax.numpy as jnp
from jax import lax
from jax.experimental import pallas as pl

N_LVL = [1563, 6250, 25000, 100000]


def _matmul_cm_nm_body(x_ref, w_ref, o_ref):
    o_ref[...] = lax.dot_general(
        x_ref[...], w_ref[...], (((0,), (1,)), ((), ())),
        preferred_element_type=jnp.float32)


def _matmul_cm_nm(x_cm, w, nb=512):
    c, n = x_cm.shape
    o = w.shape[0]
    grid = (pl.cdiv(n, nb),)
    return pl.pallas_call(
        _matmul_cm_nm_body,
        grid=grid,
        in_specs=[
            pl.BlockSpec((c, nb), lambda i: (0, i)),
            pl.BlockSpec((o, c), lambda i: (0, 0)),
        ],
        out_specs=pl.BlockSpec((nb, o), lambda i: (i, 0)),
        out_shape=jax.ShapeDtypeStruct((n, o), jnp.float32),
    )(x_cm, w)


def _segsum(rows, dst, n):
    return jax.ops.segment_sum(rows, dst, num_segments=n)


def _deg(dst, n):
    return jax.ops.segment_sum(jnp.ones(dst.shape, jnp.float32), dst, n)


def _pool_cm(x_cm, cluster, n_coarse):
    return jnp.maximum(
        jax.ops.segment_max(x_cm.T, cluster, num_segments=n_coarse), 0.0).T


def _conv_enc(x_cm, Ws, Wn, b, edges, n, agg_x_nm=None):
    src, dst = edges[0], edges[1]
    deg = jnp.maximum(_deg(dst, n), 1.0)
    if agg_x_nm is not None:
        agg = _segsum(agg_x_nm[src], dst, n) / deg[:, None]
        neigh = Wn @ agg.T
    else:
        h_nm = _matmul_cm_nm(x_cm, Wn)
        agg = _segsum(h_nm[src], dst, n) / deg[:, None]
        neigh = agg.T
    return jax.nn.relu(Ws @ x_cm + neigh + b[:, None])


def _conv_dec(u_nm, skip_cm, Ws, Wn, b, edges, n, c1):
    src, dst = edges[0], edges[1]
    Wna, Wnb = Wn[:, :c1], Wn[:, c1:]
    Wsa, Wsb = Ws[:, :c1], Ws[:, c1:]
    deg = jnp.maximum(_deg(dst, n), 1.0)
    h_nm = u_nm @ Wna.T + skip_cm.T @ Wnb.T
    agg = _segsum(h_nm[src], dst, n) / deg[:, None]
    s_nm = u_nm @ Wsa.T + skip_cm.T @ Wsb.T
    return jax.nn.relu(s_nm + agg + b[None, :])


def kernel(features, enc0_Ws, enc0_Wn, enc0_b, enc1_Ws, enc1_Wn, enc1_b,
           enc2_Ws, enc2_Wn, enc2_b, ubend_Ws, ubend_Wn, ubend_b,
           dec0_Ws, dec0_Wn, dec0_b, dec1_Ws, dec1_Wn, dec1_b,
           dec2_Ws, dec2_Wn, dec2_b, edge_index_0, edge_index_1,
           edge_index_2, edge_index_3, cluster_1, cluster_2, cluster_3):
    n0, n1, n2, n3 = N_LVL

    x3e = _conv_enc(features, enc0_Ws, enc0_Wn, enc0_b, edge_index_3, n3)
    p3 = _pool_cm(x3e, cluster_3, n2)
    x2e = _conv_enc(p3, enc1_Ws, enc1_Wn, enc1_b, edge_index_2, n2,
                    agg_x_nm=p3.T)
    p2 = _pool_cm(x2e, cluster_2, n1)
    x1e = _conv_enc(p2, enc2_Ws, enc2_Wn, enc2_b, edge_index_1, n1,
                    agg_x_nm=p2.T)
    p1 = _pool_cm(x1e, cluster_1, n0)
    xu = _conv_enc(p1, ubend_Ws, ubend_Wn, ubend_b, edge_index_0, n0,
                   agg_x_nm=p1.T)

    u0 = xu.T[cluster_1]
    d1 = _conv_dec(u0, x1e, dec0_Ws, dec0_Wn, dec0_b, edge_index_1, n1, 256)
    u1 = d1[cluster_2]
    d2 = _conv_dec(u1, x2e, dec1_Ws, dec1_Wn, dec1_b, edge_index_2, n2, 128)
    u2 = d2[cluster_3]
    out = _conv_dec(u2, x3e, dec2_Ws, dec2_Wn, dec2_b, edge_index_3, n3, 64)
    return out.T


# speedup vs baseline: 3.5564x; 3.0629x over previous
import functools

import jax
import jax.numpy as jnp
from jax import lax
from jax.experimental import pallas as pl
from jax.experimental.pallas import tpu as pltpu
from jax.experimental.pallas import tpu_sc as plsc

N_LVL = [1563, 6250, 25000, 100000]

_EB = 128
_NTILES = 16
_EPAD = _EB * _NTILES


def _round_up(x, m):
    return (x + m - 1) // m * m


def _make_sc_agg(n, n_pad, k2, e_pad, compute_deg):
    blocks_per_tile = e_pad // (_EB * _NTILES)
    rows_per_tile = n_pad // _NTILES
    zr = 128
    n_zdma = rows_per_tile // zr
    mesh = plsc.VectorSubcoreMesh(core_axis_name="c", subcore_axis_name="s")

    out_type = [jax.ShapeDtypeStruct((2, n_pad, k2), jnp.float32)]
    if compute_deg:
        out_type.append(jax.ShapeDtypeStruct((2, n_pad), jnp.float32))
    scratch = [
        pltpu.VMEM((_EB,), jnp.int32),
        pltpu.VMEM((_EB,), jnp.int32),
        pltpu.VMEM((_EB, k2), jnp.float32),
        pltpu.VMEM((zr, k2), jnp.float32),
        pltpu.SemaphoreType.DMA,
        pltpu.VMEM_SHARED((n_pad, k2), jnp.float32),
    ]
    if compute_deg:
        scratch += [
            pltpu.VMEM((_EB,), jnp.float32),
            pltpu.VMEM((zr * k2,), jnp.float32),
            pltpu.VMEM_SHARED((n_pad,), jnp.float32),
        ]

    def body(h3, src, dst, *outs_scratch):
        if compute_deg:
            (agg_out, deg_out, srcv, dstv, rows, stage, sem, acc, onesv,
             dstage, dacc) = outs_scratch
        else:
            agg_out, srcv, dstv, rows, stage, sem, acc = outs_scratch
        cid = lax.axis_index("c")
        sid = lax.axis_index("s")
        r0 = sid * rows_per_tile

        def zstage(i, _):
            stage[i // (k2 // 16), pl.ds((i % (k2 // 16)) * 16, 16)] = (
                jnp.zeros((16,), jnp.float32))
            return _
        lax.fori_loop(0, zr * (k2 // 16), zstage, None)

        def zdma(k, _):
            pltpu.sync_copy(stage, acc.at[pl.ds(r0 + k * zr, zr), :])
            return _
        lax.fori_loop(0, n_zdma, zdma, None)

        if compute_deg:
            def zdeg(i, _):
                dstage[pl.ds(i * 16, 16)] = jnp.zeros((16,), jnp.float32)
                return _
            lax.fori_loop(0, zr * k2 // 16, zdeg, None)
            dz = zr * k2
            pos = 0
            while pos < rows_per_tile:
                c = min(dz, rows_per_tile - pos)
                pltpu.sync_copy(dstage.at[pl.ds(0, c)],
                                dacc.at[pl.ds(r0 + pos, c)])
                pos += c
            def onesf(i, _):
                onesv[pl.ds(i * 16, 16)] = jnp.ones((16,), jnp.float32)
                return _
            lax.fori_loop(0, _EB // 16, onesf, None)

        plsc.subcore_barrier()

        hview = h3.at[cid]
        e0 = sid * blocks_per_tile * _EB

        def blk(j, _):
            off = e0 + j * _EB
            pltpu.sync_copy(src.at[pl.ds(off, _EB)], srcv)
            pltpu.sync_copy(dst.at[pl.ds(off, _EB)], dstv)
            pltpu.async_copy(hview.at[srcv], rows, sem).wait()
            pltpu.sync_copy(rows, acc.at[dstv], add=True)
            if compute_deg:
                @pl.when((j % 2) == cid)
                def _deg_blk():
                    pltpu.sync_copy(onesv, dacc.at[dstv], add=True)
            return _
        lax.fori_loop(0, blocks_per_tile, blk, None)

        plsc.subcore_barrier()

        aview = agg_out.at[cid]

        def wo(k, _):
            pltpu.sync_copy(acc.at[pl.ds(r0 + k * zr, zr), :], stage)
            pltpu.sync_copy(stage, aview.at[pl.ds(r0 + k * zr, zr), :])
            return _
        lax.fori_loop(0, n_zdma, wo, None)

        if compute_deg:
            dview = deg_out.at[cid]
            pos = 0
            while pos < rows_per_tile:
                c = min(zr * k2, rows_per_tile - pos)
                pltpu.sync_copy(dacc.at[pl.ds(r0 + pos, c)],
                                dstage.at[pl.ds(0, c)])
                pltpu.sync_copy(dstage.at[pl.ds(0, c)],
                                dview.at[pl.ds(r0 + pos, c)])
                pos += c

    return pl.kernel(body, out_type=tuple(out_type), mesh=mesh,
                     scratch_types=scratch,
                     compiler_params=pltpu.CompilerParams(
                         use_tc_tiling_on_sc=False))


def _sc_agg(h_nm, src_pad, dst_pad, n, compute_deg):
    nn, k = h_nm.shape
    k2 = k // 2
    n_pad = _round_up(max(n + 8, nn), 2048)
    h3 = jnp.stack((h_nm[:, :k2], h_nm[:, k2:]))
    if n_pad != nn:
        h3 = jnp.pad(h3, ((0, 0), (0, n_pad - nn), (0, 0)))
    fn = _make_sc_agg(n, n_pad, k2, src_pad.shape[0], compute_deg)
    if compute_deg:
        agg3, degp = fn(h3, src_pad, dst_pad)
        deg = (degp[0] + degp[1])[:n]
    else:
        (agg3,) = fn(h3, src_pad, dst_pad)
        deg = None
    agg = jnp.concatenate((agg3[0, :n], agg3[1, :n]), axis=1)
    return agg, deg


def _pad_edges(edge_index, n):
    e = edge_index.shape[1]
    ep = _round_up(e, _EPAD)
    src = jnp.concatenate(
        (edge_index[0], jnp.zeros((ep - e,), jnp.int32)))
    dst = jnp.concatenate(
        (edge_index[1], jnp.full((ep - e,), n, jnp.int32)))
    return src, dst


def _matmul_cm_nm_body(x_ref, w_ref, o_ref):
    o_ref[...] = lax.dot_general(
        x_ref[...], w_ref[...], (((0,), (1,)), ((), ())),
        preferred_element_type=jnp.float32)


def _matmul_cm_nm(x_cm, w, nb=512):
    c, n = x_cm.shape
    o = w.shape[0]
    grid = (pl.cdiv(n, nb),)
    return pl.pallas_call(
        _matmul_cm_nm_body,
        grid=grid,
        in_specs=[
            pl.BlockSpec((c, nb), lambda i: (0, i)),
            pl.BlockSpec((o, c), lambda i: (0, 0)),
        ],
        out_specs=pl.BlockSpec((nb, o), lambda i: (i, 0)),
        out_shape=jax.ShapeDtypeStruct((n, o), jnp.float32),
    )(x_cm, w)


def _segsum(rows, dst, n):
    return jax.ops.segment_sum(rows, dst, num_segments=n)


def _deg(dst, n):
    return jax.ops.segment_sum(jnp.ones(dst.shape, jnp.float32), dst, n)


def _pool_cm(x_cm, cluster, n_coarse):
    return jnp.maximum(
        jax.ops.segment_max(x_cm.T, cluster, num_segments=n_coarse), 0.0).T


def _conv_enc(x_cm, Ws, Wn, b, edges_pad, n, agg_x_nm=None):
    src, dst = edges_pad
    if agg_x_nm is not None:
        agg, deg = _sc_agg(agg_x_nm, src, dst, n, True)
        deg = jnp.maximum(deg, 1.0)
        neigh = Wn @ (agg / deg[:, None]).T
    else:
        h_nm = _matmul_cm_nm(x_cm, Wn)
        agg, deg = _sc_agg(h_nm, src, dst, n, True)
        deg = jnp.maximum(deg, 1.0)
        neigh = (agg / deg[:, None]).T
    return jax.nn.relu(Ws @ x_cm + neigh + b[:, None]), deg


def _conv_dec(u_nm, skip_cm, Ws, Wn, b, edges_pad, n, c1, deg):
    src, dst = edges_pad
    Wna, Wnb = Wn[:, :c1], Wn[:, c1:]
    Wsa, Wsb = Ws[:, :c1], Ws[:, c1:]
    h_nm = u_nm @ Wna.T + skip_cm.T @ Wnb.T
    agg, _ = _sc_agg(h_nm, src, dst, n, False)
    s_nm = u_nm @ Wsa.T + skip_cm.T @ Wsb.T
    return jax.nn.relu(s_nm + agg / deg[:, None] + b[None, :])


def kernel(features, enc0_Ws, enc0_Wn, enc0_b, enc1_Ws, enc1_Wn, enc1_b,
           enc2_Ws, enc2_Wn, enc2_b, ubend_Ws, ubend_Wn, ubend_b,
           dec0_Ws, dec0_Wn, dec0_b, dec1_Ws, dec1_Wn, dec1_b,
           dec2_Ws, dec2_Wn, dec2_b, edge_index_0, edge_index_1,
           edge_index_2, edge_index_3, cluster_1, cluster_2, cluster_3):
    n0, n1, n2, n3 = N_LVL
    e3 = _pad_edges(edge_index_3, n3)
    e2 = _pad_edges(edge_index_2, n2)
    e1 = _pad_edges(edge_index_1, n1)
    e0 = _pad_edges(edge_index_0, n0)

    x3e, deg3 = _conv_enc(features, enc0_Ws, enc0_Wn, enc0_b, e3, n3)
    p3 = _pool_cm(x3e, cluster_3, n2)
    x2e, deg2 = _conv_enc(p3, enc1_Ws, enc1_Wn, enc1_b, e2, n2,
                          agg_x_nm=p3.T)
    p2 = _pool_cm(x2e, cluster_2, n1)
    x1e, deg1 = _conv_enc(p2, enc2_Ws, enc2_Wn, enc2_b, e1, n1,
                          agg_x_nm=p2.T)
    p1 = _pool_cm(x1e, cluster_1, n0)
    xu, _ = _conv_enc(p1, ubend_Ws, ubend_Wn, ubend_b, e0, n0,
                      agg_x_nm=p1.T)

    u0 = xu.T[cluster_1]
    d1 = _conv_dec(u0, x1e, dec0_Ws, dec0_Wn, dec0_b, e1, n1, 256, deg1)
    u1 = d1[cluster_2]
    d2 = _conv_dec(u1, x2e, dec1_Ws, dec1_Wn, dec1_b, e2, n2, 128, deg2)
    u2 = d2[cluster_3]
    out = _conv_dec(u2, x3e, dec2_Ws, dec2_Wn, dec2_b, e3, n3, 64, deg3)
    return out.T


# speedup vs baseline: 5.0466x; 1.4190x over previous
import functools

import jax
import jax.numpy as jnp
from jax import lax
from jax.experimental import pallas as pl
from jax.experimental.pallas import tpu as pltpu
from jax.experimental.pallas import tpu_sc as plsc

N_LVL = [1563, 6250, 25000, 100000]

_EB = 128
_G_CAP = 8
_NTILES = 16
_EPAD = _EB * _NTILES


def _round_up(x, m):
    return (x + m - 1) // m * m


def _make_sc_agg(n, n_pad, k2, e_pad, compute_deg):
    blocks_per_tile = e_pad // (_EB * _NTILES)
    rows_per_tile = n_pad // _NTILES
    zr = 128
    n_zdma = rows_per_tile // zr
    g = min(_G_CAP, 256 // k2)
    n_sb = blocks_per_tile // g
    n_rem = blocks_per_tile % g
    mesh = plsc.VectorSubcoreMesh(core_axis_name="c", subcore_axis_name="s")

    out_type = [jax.ShapeDtypeStruct((2, n_pad, k2), jnp.float32)]
    if compute_deg:
        out_type.append(jax.ShapeDtypeStruct((2, n_pad), jnp.float32))
    scratch = [
        pltpu.VMEM((g, _EB), jnp.int32),
        pltpu.VMEM((g, _EB), jnp.int32),
        pltpu.VMEM((g * _EB, k2), jnp.float32),
        pltpu.VMEM((zr, k2), jnp.float32),
        pltpu.SemaphoreType.DMA,
        pltpu.SemaphoreType.DMA,
        pltpu.VMEM_SHARED((n_pad, k2), jnp.float32),
    ]
    if compute_deg:
        scratch += [
            pltpu.VMEM((_EB,), jnp.float32),
            pltpu.VMEM((zr * k2,), jnp.float32),
            pltpu.VMEM_SHARED((n_pad,), jnp.float32),
        ]

    def body(h3, src, dst, *outs_scratch):
        if compute_deg:
            (agg_out, deg_out, srcv, dstv, rows, stage, sem, sems, acc,
             onesv, dstage, dacc) = outs_scratch
        else:
            agg_out, srcv, dstv, rows, stage, sem, sems, acc = outs_scratch
        cid = lax.axis_index("c")
        sid = lax.axis_index("s")
        r0 = sid * rows_per_tile

        def zstage(i, _):
            stage[i // (k2 // 16), pl.ds((i % (k2 // 16)) * 16, 16)] = (
                jnp.zeros((16,), jnp.float32))
            return _
        lax.fori_loop(0, zr * (k2 // 16), zstage, None)

        def zdma(k, _):
            pltpu.sync_copy(stage, acc.at[pl.ds(r0 + k * zr, zr), :])
            return _
        lax.fori_loop(0, n_zdma, zdma, None)

        if compute_deg:
            def zdeg(i, _):
                dstage[pl.ds(i * 16, 16)] = jnp.zeros((16,), jnp.float32)
                return _
            lax.fori_loop(0, zr * k2 // 16, zdeg, None)
            dz = zr * k2
            pos = 0
            while pos < rows_per_tile:
                c = min(dz, rows_per_tile - pos)
                pltpu.sync_copy(dstage.at[pl.ds(0, c)],
                                dacc.at[pl.ds(r0 + pos, c)])
                pos += c
            def onesf(i, _):
                onesv[pl.ds(i * 16, 16)] = jnp.full((16,), 0.5, jnp.float32)
                return _
            lax.fori_loop(0, _EB // 16, onesf, None)

        plsc.subcore_barrier()

        hview = h3.at[cid]
        row0 = sid * blocks_per_tile

        def do_superblock(base_blk, gg):
            base_e = base_blk * _EB
            descs = [
                pltpu.async_copy(src.at[pl.ds(base_e + q * _EB, _EB)],
                                 srcv.at[q], sem)
                for q in range(gg)]
            descs += [
                pltpu.async_copy(dst.at[pl.ds(base_e + q * _EB, _EB)],
                                 dstv.at[q], sem)
                for q in range(gg)]
            for dsc in descs:
                dsc.wait()
            descs = [
                pltpu.async_copy(hview.at[srcv.at[q]],
                                 rows.at[pl.ds(q * _EB, _EB), :], sem)
                for q in range(gg)]
            for dsc in descs:
                dsc.wait()
            descs = []
            for q in range(gg):
                descs.append(pltpu.async_copy(
                    rows.at[pl.ds(q * _EB, _EB), :], acc.at[dstv.at[q]],
                    sems, add=True))
                if compute_deg:
                    descs.append(pltpu.async_copy(
                        onesv, dacc.at[dstv.at[q]], sems, add=True))
            for dsc in descs:
                dsc.wait()

        def sb_body(j, _):
            do_superblock(row0 + j * g, g)
            return _
        lax.fori_loop(0, n_sb, sb_body, None)
        if n_rem:
            do_superblock(row0 + n_sb * g, n_rem)

        plsc.subcore_barrier()

        aview = agg_out.at[cid]

        def wo(k, _):
            pltpu.sync_copy(acc.at[pl.ds(r0 + k * zr, zr), :], stage)
            pltpu.sync_copy(stage, aview.at[pl.ds(r0 + k * zr, zr), :])
            return _
        lax.fori_loop(0, n_zdma, wo, None)

        if compute_deg:
            dview = deg_out.at[cid]
            pos = 0
            while pos < rows_per_tile:
                c = min(zr * k2, rows_per_tile - pos)
                pltpu.sync_copy(dacc.at[pl.ds(r0 + pos, c)],
                                dstage.at[pl.ds(0, c)])
                pltpu.sync_copy(dstage.at[pl.ds(0, c)],
                                dview.at[pl.ds(r0 + pos, c)])
                pos += c

    return pl.kernel(body, out_type=tuple(out_type), mesh=mesh,
                     scratch_types=scratch,
                     compiler_params=pltpu.CompilerParams(
                         use_tc_tiling_on_sc=False))


def _sc_agg(h_nm, src_pad, dst_pad, n, compute_deg):
    nn, k = h_nm.shape
    k2 = k // 2
    n_pad = _round_up(max(n + 8, nn), 2048)
    h3 = jnp.stack((h_nm[:, :k2], h_nm[:, k2:]))
    if n_pad != nn:
        h3 = jnp.pad(h3, ((0, 0), (0, n_pad - nn), (0, 0)))
    fn = _make_sc_agg(n, n_pad, k2, src_pad.size, compute_deg)
    if compute_deg:
        agg3, degp = fn(h3, src_pad, dst_pad)
        deg = (degp[0] + degp[1])[:n]
    else:
        (agg3,) = fn(h3, src_pad, dst_pad)
        deg = None
    agg = jnp.concatenate((agg3[0, :n], agg3[1, :n]), axis=1)
    return agg, deg


def _pad_edges(edge_index, n):
    e = edge_index.shape[1]
    ep = _round_up(e, _EPAD)
    src = jnp.concatenate(
        (edge_index[0], jnp.zeros((ep - e,), jnp.int32)))
    dst = jnp.concatenate(
        (edge_index[1], jnp.full((ep - e,), n, jnp.int32)))
    return src, dst


def _matmul_cm_nm_body(x_ref, w_ref, o_ref):
    o_ref[...] = lax.dot_general(
        x_ref[...], w_ref[...], (((0,), (1,)), ((), ())),
        preferred_element_type=jnp.float32)


def _matmul_cm_nm(x_cm, w, nb=512):
    c, n = x_cm.shape
    o = w.shape[0]
    grid = (pl.cdiv(n, nb),)
    return pl.pallas_call(
        _matmul_cm_nm_body,
        grid=grid,
        in_specs=[
            pl.BlockSpec((c, nb), lambda i: (0, i)),
            pl.BlockSpec((o, c), lambda i: (0, 0)),
        ],
        out_specs=pl.BlockSpec((nb, o), lambda i: (i, 0)),
        out_shape=jax.ShapeDtypeStruct((n, o), jnp.float32),
    )(x_cm, w)


def _segsum(rows, dst, n):
    return jax.ops.segment_sum(rows, dst, num_segments=n)


def _deg(dst, n):
    return jax.ops.segment_sum(jnp.ones(dst.shape, jnp.float32), dst, n)


def _pool_cm(x_cm, cluster, n_coarse):
    return jnp.maximum(
        jax.ops.segment_max(x_cm.T, cluster, num_segments=n_coarse), 0.0).T


def _conv_enc(x_cm, Ws, Wn, b, edges_pad, n, agg_x_nm=None):
    src, dst = edges_pad
    if agg_x_nm is not None:
        agg, deg = _sc_agg(agg_x_nm, src, dst, n, True)
        deg = jnp.maximum(deg, 1.0)
        neigh = Wn @ (agg / deg[:, None]).T
    else:
        h_nm = _matmul_cm_nm(x_cm, Wn)
        agg, deg = _sc_agg(h_nm, src, dst, n, True)
        deg = jnp.maximum(deg, 1.0)
        neigh = (agg / deg[:, None]).T
    return jax.nn.relu(Ws @ x_cm + neigh + b[:, None]), deg


def _conv_dec(u_nm, skip_cm, Ws, Wn, b, edges_pad, n, c1, deg):
    src, dst = edges_pad
    Wna, Wnb = Wn[:, :c1], Wn[:, c1:]
    Wsa, Wsb = Ws[:, :c1], Ws[:, c1:]
    h_nm = u_nm @ Wna.T + skip_cm.T @ Wnb.T
    agg, _ = _sc_agg(h_nm, src, dst, n, False)
    s_nm = u_nm @ Wsa.T + skip_cm.T @ Wsb.T
    return jax.nn.relu(s_nm + agg / deg[:, None] + b[None, :])


def kernel(features, enc0_Ws, enc0_Wn, enc0_b, enc1_Ws, enc1_Wn, enc1_b,
           enc2_Ws, enc2_Wn, enc2_b, ubend_Ws, ubend_Wn, ubend_b,
           dec0_Ws, dec0_Wn, dec0_b, dec1_Ws, dec1_Wn, dec1_b,
           dec2_Ws, dec2_Wn, dec2_b, edge_index_0, edge_index_1,
           edge_index_2, edge_index_3, cluster_1, cluster_2, cluster_3):
    n0, n1, n2, n3 = N_LVL
    e3 = _pad_edges(edge_index_3, n3)
    e2 = _pad_edges(edge_index_2, n2)
    e1 = _pad_edges(edge_index_1, n1)
    e0 = _pad_edges(edge_index_0, n0)

    x3e, deg3 = _conv_enc(features, enc0_Ws, enc0_Wn, enc0_b, e3, n3)
    p3 = _pool_cm(x3e, cluster_3, n2)
    x2e, deg2 = _conv_enc(p3, enc1_Ws, enc1_Wn, enc1_b, e2, n2,
                          agg_x_nm=p3.T)
    p2 = _pool_cm(x2e, cluster_2, n1)
    x1e, deg1 = _conv_enc(p2, enc2_Ws, enc2_Wn, enc2_b, e1, n1,
                          agg_x_nm=p2.T)
    p1 = _pool_cm(x1e, cluster_1, n0)
    xu, _ = _conv_enc(p1, ubend_Ws, ubend_Wn, ubend_b, e0, n0,
                      agg_x_nm=p1.T)

    u0 = xu.T[cluster_1]
    d1 = _conv_dec(u0, x1e, dec0_Ws, dec0_Wn, dec0_b, e1, n1, 256, deg1)
    u1 = d1[cluster_2]
    d2 = _conv_dec(u1, x2e, dec1_Ws, dec1_Wn, dec1_b, e2, n2, 128, deg2)
    u2 = d2[cluster_3]
    out = _conv_dec(u2, x3e, dec2_Ws, dec2_Wn, dec2_b, e3, n3, 64, deg3)
    return out.T


# speedup vs baseline: 5.1324x; 1.0170x over previous
import functools

import jax
import jax.numpy as jnp
from jax import lax
from jax.experimental import pallas as pl
from jax.experimental.pallas import tpu as pltpu
from jax.experimental.pallas import tpu_sc as plsc

N_LVL = [1563, 6250, 25000, 100000]

_EB = 128
_G_CAP = 8
_NTILES = 16
_EPAD = _EB * _NTILES


def _round_up(x, m):
    return (x + m - 1) // m * m


def _make_sc_agg(n, n_pad, k2, e_pad, compute_deg):
    blocks_per_tile = e_pad // (_EB * _NTILES)
    rows_per_tile = n_pad // _NTILES
    zr = 128
    n_zdma = rows_per_tile // zr
    g = min(_G_CAP, 256 // k2)
    n_sb = blocks_per_tile // g
    n_rem = blocks_per_tile % g
    mesh = plsc.VectorSubcoreMesh(core_axis_name="c", subcore_axis_name="s")

    out_type = [jax.ShapeDtypeStruct((2, n_pad, k2), jnp.float32)]
    if compute_deg:
        out_type.append(jax.ShapeDtypeStruct((2, n_pad), jnp.float32))
    scratch = [
        pltpu.VMEM((g, _EB), jnp.int32),
        pltpu.VMEM((g, _EB), jnp.int32),
        pltpu.VMEM((g * _EB, k2), jnp.float32),
        pltpu.VMEM((zr, k2), jnp.float32),
        pltpu.SemaphoreType.DMA,
        pltpu.SemaphoreType.DMA,
        pltpu.VMEM_SHARED((n_pad, k2), jnp.float32),
    ]
    if compute_deg:
        scratch += [
            pltpu.VMEM((_EB,), jnp.float32),
            pltpu.VMEM((zr * k2,), jnp.float32),
            pltpu.VMEM_SHARED((n_pad,), jnp.float32),
        ]

    def body(h3, src, dst, *outs_scratch):
        if compute_deg:
            (agg_out, deg_out, srcv, dstv, rows, stage, sem, sems, acc,
             onesv, dstage, dacc) = outs_scratch
        else:
            agg_out, srcv, dstv, rows, stage, sem, sems, acc = outs_scratch
        cid = lax.axis_index("c")
        sid = lax.axis_index("s")
        r0 = sid * rows_per_tile

        def zstage(i, _):
            stage[i // (k2 // 16), pl.ds((i % (k2 // 16)) * 16, 16)] = (
                jnp.zeros((16,), jnp.float32))
            return _
        lax.fori_loop(0, zr * (k2 // 16), zstage, None)

        def zdma(k, _):
            pltpu.sync_copy(stage, acc.at[pl.ds(r0 + k * zr, zr), :])
            return _
        lax.fori_loop(0, n_zdma, zdma, None)

        if compute_deg:
            def zdeg(i, _):
                dstage[pl.ds(i * 16, 16)] = jnp.zeros((16,), jnp.float32)
                return _
            lax.fori_loop(0, zr * k2 // 16, zdeg, None)
            dz = zr * k2
            pos = 0
            while pos < rows_per_tile:
                c = min(dz, rows_per_tile - pos)
                pltpu.sync_copy(dstage.at[pl.ds(0, c)],
                                dacc.at[pl.ds(r0 + pos, c)])
                pos += c
            def onesf(i, _):
                onesv[pl.ds(i * 16, 16)] = jnp.full((16,), 0.5, jnp.float32)
                return _
            lax.fori_loop(0, _EB // 16, onesf, None)

        plsc.subcore_barrier()

        hview = h3.at[cid]
        row0 = sid * blocks_per_tile

        def do_superblock(base_blk, gg):
            base_e = base_blk * _EB
            descs = [
                pltpu.async_copy(src.at[pl.ds(base_e + q * _EB, _EB)],
                                 srcv.at[q], sem)
                for q in range(gg)]
            descs += [
                pltpu.async_copy(dst.at[pl.ds(base_e + q * _EB, _EB)],
                                 dstv.at[q], sem)
                for q in range(gg)]
            for dsc in descs:
                dsc.wait()
            descs = [
                pltpu.async_copy(hview.at[srcv.at[q]],
                                 rows.at[pl.ds(q * _EB, _EB), :], sem)
                for q in range(gg)]
            for dsc in descs:
                dsc.wait()
            descs = []
            for q in range(gg):
                descs.append(pltpu.async_copy(
                    rows.at[pl.ds(q * _EB, _EB), :], acc.at[dstv.at[q]],
                    sems, add=True))
                if compute_deg:
                    descs.append(pltpu.async_copy(
                        onesv, dacc.at[dstv.at[q]], sems, add=True))
            for dsc in descs:
                dsc.wait()

        def sb_body(j, _):
            do_superblock(row0 + j * g, g)
            return _
        lax.fori_loop(0, n_sb, sb_body, None)
        if n_rem:
            do_superblock(row0 + n_sb * g, n_rem)

        plsc.subcore_barrier()

        aview = agg_out.at[cid]

        def wo(k, _):
            pltpu.sync_copy(acc.at[pl.ds(r0 + k * zr, zr), :], stage)
            pltpu.sync_copy(stage, aview.at[pl.ds(r0 + k * zr, zr), :])
            return _
        lax.fori_loop(0, n_zdma, wo, None)

        if compute_deg:
            dview = deg_out.at[cid]
            pos = 0
            while pos < rows_per_tile:
                c = min(zr * k2, rows_per_tile - pos)
                pltpu.sync_copy(dacc.at[pl.ds(r0 + pos, c)],
                                dstage.at[pl.ds(0, c)])
                pltpu.sync_copy(dstage.at[pl.ds(0, c)],
                                dview.at[pl.ds(r0 + pos, c)])
                pos += c

    return pl.kernel(body, out_type=tuple(out_type), mesh=mesh,
                     scratch_types=scratch,
                     compiler_params=pltpu.CompilerParams(
                         use_tc_tiling_on_sc=False))


def _sc_agg(h_nm, src_pad, dst_pad, n, compute_deg):
    nn, k = h_nm.shape
    k2 = k // 2
    n_pad = _round_up(max(n + 8, nn), 2048)
    h3 = jnp.stack((h_nm[:, :k2], h_nm[:, k2:]))
    if n_pad != nn:
        h3 = jnp.pad(h3, ((0, 0), (0, n_pad - nn), (0, 0)))
    fn = _make_sc_agg(n, n_pad, k2, src_pad.size, compute_deg)
    if compute_deg:
        agg3, degp = fn(h3, src_pad, dst_pad)
        deg = degp[0] + degp[1]
    else:
        (agg3,) = fn(h3, src_pad, dst_pad)
        deg = None
    agg = jnp.concatenate((agg3[0], agg3[1]), axis=1)
    return agg, deg


def _make_sc_unpool(nf_pad, k, nc):
    g = max(1, min(_G_CAP, 256 // k))
    n_blocks = nf_pad // _EB
    qb = -(-n_blocks // 32)
    mesh = plsc.VectorSubcoreMesh(core_axis_name="c", subcore_axis_name="s")

    scratch = [
        pltpu.VMEM((g, _EB), jnp.int32),
        pltpu.VMEM((g * _EB, k), jnp.float32),
        pltpu.SemaphoreType.DMA,
    ]

    def body(table, cluster, out, idxv, rows, sem):
        cid = lax.axis_index("c")
        sid = lax.axis_index("s")
        wid = sid * 2 + cid
        b0 = wid * qb
        nb = jnp.clip(n_blocks - b0, 0, qb)

        def do_sb(base_blk, gg):
            descs = [
                pltpu.async_copy(cluster.at[pl.ds((base_blk + q) * _EB, _EB)],
                                 idxv.at[q], sem)
                for q in range(gg)]
            for dsc in descs:
                dsc.wait()
            descs = [
                pltpu.async_copy(table.at[idxv.at[q]],
                                 rows.at[pl.ds(q * _EB, _EB), :], sem)
                for q in range(gg)]
            for dsc in descs:
                dsc.wait()
            descs = [
                pltpu.async_copy(rows.at[pl.ds(q * _EB, _EB), :],
                                 out.at[pl.ds((base_blk + q) * _EB, _EB), :],
                                 sem)
                for q in range(gg)]
            for dsc in descs:
                dsc.wait()

        n_sb = nb // g

        def sb_body(j, _):
            do_sb(b0 + j * g, g)
            return _
        lax.fori_loop(0, n_sb, sb_body, None)

        def rem_body(j, _):
            do_sb(b0 + j, 1)
            return _
        lax.fori_loop(n_sb * g, nb, rem_body, None)

    return pl.kernel(
        body,
        out_type=jax.ShapeDtypeStruct((nf_pad, k), jnp.float32),
        mesh=mesh, scratch_types=scratch,
        compiler_params=pltpu.CompilerParams(use_tc_tiling_on_sc=False))


def _sc_unpool(table_nm, cluster, nf):
    nc, k = table_nm.shape
    nf_pad = _round_up(nf, 2048)
    cl = jnp.concatenate(
        (cluster, jnp.zeros((nf_pad - nf,), jnp.int32)))
    return _make_sc_unpool(nf_pad, k, nc)(table_nm, cl)


def _make_sc_pool(k, nf_pad, nc, nc_pad):
    cpw = k // 32
    cb = 2048
    n_chunks = nf_pad // cb
    acc_n = nc_pad + 2048
    mesh = plsc.VectorSubcoreMesh(core_axis_name="c", subcore_axis_name="s")

    scratch = [
        pltpu.VMEM((cb,), jnp.int32),
        pltpu.VMEM((cpw, cb), jnp.float32),
        pltpu.VMEM((cpw * acc_n,), jnp.float32),
        pltpu.SemaphoreType.DMA,
    ]

    def body(x_cm, cluster, p_out, cluv, xbuf, acc, sem):
        cid = lax.axis_index("c")
        sid = lax.axis_index("s")
        wid = sid * 2 + cid
        ch0 = wid * cpw

        def zf(i, _):
            acc[pl.ds(i * 16, 16)] = jnp.zeros((16,), jnp.float32)
            return _
        lax.fori_loop(0, cpw * acc_n // 16, zf, None)

        def chunk(c, _):
            pos = c * cb
            descs = [pltpu.async_copy(cluster.at[pl.ds(pos, cb)], cluv, sem)]
            descs += [
                pltpu.async_copy(x_cm.at[ch0 + ci].at[pl.ds(pos, cb)],
                                 xbuf.at[ci], sem)
                for ci in range(cpw)]
            for dsc in descs:
                dsc.wait()

            def vec(j, _):
                idx = cluv[pl.ds(j * 16, 16)]
                for ci in range(cpw):
                    idxo = idx + ci * acc_n
                    val = xbuf[ci, pl.ds(j * 16, 16)]
                    cur = plsc.load_gather(acc, [idxo])
                    new = jnp.maximum(cur, val)
                    plsc.store_scatter(acc, [idxo], new)

                    def cond(carry):
                        _, lost = carry
                        return jnp.any(lost)

                    def retry(carry):
                        new, lost = carry
                        cur = plsc.load_gather(acc, [idxo])
                        new = jnp.maximum(cur, new)
                        plsc.store_scatter(acc, [idxo], new, mask=lost)
                        back = plsc.load_gather(acc, [idxo])
                        return new, back < new

                    back = plsc.load_gather(acc, [idxo])
                    lax.while_loop(cond, retry, (new, back < new))
                return _
            lax.fori_loop(0, cb // 16, vec, None)
            return _
        lax.fori_loop(0, n_chunks, chunk, None)

        for ci in range(cpw):
            pltpu.sync_copy(acc.at[pl.ds(ci * acc_n, nc_pad)],
                            p_out.at[ch0 + ci])

    return pl.kernel(
        body,
        out_type=jax.ShapeDtypeStruct((k, nc_pad), jnp.float32),
        mesh=mesh, scratch_types=scratch,
        compiler_params=pltpu.CompilerParams(needs_layout_passes=False))


def _sc_pool(x_cm_pad, cluster, nc):
    k, nf_pad = x_cm_pad.shape
    nc_pad = _round_up(nc, 2048)
    cl = jnp.concatenate(
        (cluster, jnp.full((nf_pad - cluster.shape[0],), nc, jnp.int32)))
    return _make_sc_pool(k, nf_pad, nc, nc_pad)(x_cm_pad, cl)


def _pad_edges(edge_index, n):
    e = edge_index.shape[1]
    ep = _round_up(e, _EPAD)
    src = jnp.concatenate(
        (edge_index[0], jnp.zeros((ep - e,), jnp.int32)))
    dst = jnp.concatenate(
        (edge_index[1], jnp.full((ep - e,), n, jnp.int32)))
    return src, dst


def _matmul_cm_nm_body(x_ref, w_ref, o_ref):
    o_ref[...] = lax.dot_general(
        x_ref[...], w_ref[...], (((0,), (1,)), ((), ())),
        preferred_element_type=jnp.float32)


def _matmul_cm_nm(x_cm, w, nb=512):
    c, n = x_cm.shape
    o = w.shape[0]
    grid = (pl.cdiv(n, nb),)
    return pl.pallas_call(
        _matmul_cm_nm_body,
        grid=grid,
        in_specs=[
            pl.BlockSpec((c, nb), lambda i: (0, i)),
            pl.BlockSpec((o, c), lambda i: (0, 0)),
        ],
        out_specs=pl.BlockSpec((nb, o), lambda i: (i, 0)),
        out_shape=jax.ShapeDtypeStruct((n, o), jnp.float32),
    )(x_cm, w)


def _conv_enc(x_cm, Ws, Wn, b, edges_pad, n, agg_x=False):
    src, dst = edges_pad
    if agg_x:
        agg, deg = _sc_agg(x_cm.T, src, dst, n, True)
        rdeg = 1.0 / jnp.maximum(deg, 1.0)
        neigh = Wn @ (agg * rdeg[:, None]).T
    else:
        h_nm = _matmul_cm_nm(x_cm, Wn)
        agg, deg = _sc_agg(h_nm, src, dst, n, True)
        rdeg = 1.0 / jnp.maximum(deg, 1.0)
        neigh = (agg * rdeg[:, None]).T
    s = Ws @ x_cm
    if s.shape[1] != neigh.shape[1]:
        s = jnp.pad(s, ((0, 0), (0, neigh.shape[1] - s.shape[1])))
    return jax.nn.relu(s + neigh + b[:, None]), rdeg


def _conv_dec(u_nm, skip_cm, Ws, Wn, b, edges_pad, n, c1, rdeg):
    src, dst = edges_pad
    Wna, Wnb = Wn[:, :c1], Wn[:, c1:]
    Wsa, Wsb = Ws[:, :c1], Ws[:, c1:]
    h_nm = u_nm @ Wna.T + skip_cm.T @ Wnb.T
    agg, _ = _sc_agg(h_nm, src, dst, n, False)
    s_nm = u_nm @ Wsa.T + skip_cm.T @ Wsb.T
    return jax.nn.relu(s_nm + agg * rdeg[:, None] + b[None, :])


def kernel(features, enc0_Ws, enc0_Wn, enc0_b, enc1_Ws, enc1_Wn, enc1_b,
           enc2_Ws, enc2_Wn, enc2_b, ubend_Ws, ubend_Wn, ubend_b,
           dec0_Ws, dec0_Wn, dec0_b, dec1_Ws, dec1_Wn, dec1_b,
           dec2_Ws, dec2_Wn, dec2_b, edge_index_0, edge_index_1,
           edge_index_2, edge_index_3, cluster_1, cluster_2, cluster_3):
    n0, n1, n2, n3 = N_LVL
    e3 = _pad_edges(edge_index_3, n3)
    e2 = _pad_edges(edge_index_2, n2)
    e1 = _pad_edges(edge_index_1, n1)
    e0 = _pad_edges(edge_index_0, n0)

    x3e, rdeg3 = _conv_enc(features, enc0_Ws, enc0_Wn, enc0_b, e3, n3)
    p3 = _sc_pool(x3e, cluster_3, n2)
    x2e, rdeg2 = _conv_enc(p3, enc1_Ws, enc1_Wn, enc1_b, e2, n2, agg_x=True)
    p2 = _sc_pool(x2e, cluster_2, n1)
    x1e, rdeg1 = _conv_enc(p2, enc2_Ws, enc2_Wn, enc2_b, e1, n1, agg_x=True)
    p1 = _sc_pool(x1e, cluster_1, n0)
    xu, _ = _conv_enc(p1, ubend_Ws, ubend_Wn, ubend_b, e0, n0, agg_x=True)

    u0 = _sc_unpool(xu.T, cluster_1, n1)
    d1 = _conv_dec(u0, x1e, dec0_Ws, dec0_Wn, dec0_b, e1, n1, 256, rdeg1)
    u1 = _sc_unpool(d1, cluster_2, n2)
    d2 = _conv_dec(u1, x2e, dec1_Ws, dec1_Wn, dec1_b, e2, n2, 128, rdeg2)
    u2 = _sc_unpool(d2, cluster_3, n3)
    out = _conv_dec(u2, x3e, dec2_Ws, dec2_Wn, dec2_b, e3, n3, 64, rdeg3)
    return out[:n3].T
